# baseline (speedup 1.0000x reference)
import functools

import jax
import jax.numpy as jnp
from jax import lax
from jax.experimental import pallas as pl
from jax.experimental.pallas import tpu as pltpu
from jax.experimental.pallas import tpu_sc as plsc

D = 64
DP = 128
CHUNK = 128
NBUF = 2


@functools.cache
def _make_gather(n_rows):
    info = plsc.get_sparse_core_info()
    nw = info.num_cores * info.num_subcores
    rows_per_w = n_rows // nw
    n_chunks = rows_per_w // CHUNK
    assert n_chunks * CHUNK * nw == n_rows and n_chunks % NBUF == 0

    mesh = plsc.VectorSubcoreMesh(core_axis_name="c", subcore_axis_name="s")

    @functools.partial(
        pl.kernel,
        mesh=mesh,
        out_type=jax.ShapeDtypeStruct((n_rows, DP), jnp.float32),
        scratch_types=[
            pltpu.VMEM((n_chunks, CHUNK), jnp.int32),
            pltpu.VMEM((NBUF, CHUNK, DP), jnp.float32),
            pltpu.SemaphoreType.DMA,
            pltpu.SemaphoreType.DMA,
            pltpu.SemaphoreType.DMA,
            pltpu.SemaphoreType.DMA,
        ],
        compiler_params=pltpu.CompilerParams(use_tc_tiling_on_sc=True),
    )
    def gather(idx_hbm, table_hbm, out_hbm, idx_v, rows_v, g0, g1, s0_, s1_):
        gsems = (g0, g1)
        ssems = (s0_, s1_)
        wid = lax.axis_index("s") * info.num_cores + lax.axis_index("c")
        pltpu.sync_copy(idx_hbm.at[wid], idx_v)
        base = wid * rows_per_w

        def fire_gather(g, b):
            pltpu.async_copy(table_hbm.at[idx_v.at[g]], rows_v.at[b], gsems[b])

        def drain_gather(g, b):
            pltpu.make_async_copy(
                table_hbm.at[idx_v.at[g]], rows_v.at[b], gsems[b]
            ).wait()

        def fire_store(g, b):
            row0 = pl.multiple_of(base + g * CHUNK, CHUNK)
            pltpu.async_copy(rows_v.at[b], out_hbm.at[pl.ds(row0, CHUNK)], ssems[b])

        def drain_store(g, b):
            row0 = pl.multiple_of(base + g * CHUNK, CHUNK)
            pltpu.make_async_copy(
                rows_v.at[b], out_hbm.at[pl.ds(row0, CHUNK)], ssems[b]
            ).wait()

        fire_gather(0, 0)

        @pl.loop(0, n_chunks, step=NBUF)
        def _(sg):
            drain_gather(sg, 0)

            @pl.when(sg >= 1)
            def _():
                drain_store(sg - 1, 1)

            fire_store(sg, 0)
            fire_gather(sg + 1, 1)

            drain_gather(sg + 1, 1)
            drain_store(sg, 0)
            fire_store(sg + 1, 1)

            @pl.when(sg + 2 < n_chunks)
            def _():
                fire_gather(sg + 2, 0)

        drain_store(n_chunks - 1, 1)

    return gather


@jax.jit
def kernel(idx, weight):
    b, t = idx.shape
    n = b * t
    info = plsc.get_sparse_core_info()
    nw = info.num_cores * info.num_subcores
    idx3 = idx.reshape(nw, n // (nw * CHUNK), CHUNK).astype(jnp.int32)
    wpad = jnp.pad(weight, ((0, 0), (0, DP - D)))
    out = _make_gather(n)(idx3, wpad)
    return out[:, :D].reshape(b, t, weight.shape[1])
